# baseline (speedup 1.0000x reference)
import jax
import jax.numpy as jnp
from jax.experimental import pallas as pl
from jax.experimental.pallas import tpu as pltpu

_LEVELS = 16
_OFFSETS = ((0, 5), (-5, 5), (-5, 0), (-5, -5))
_W1SUM = 728.0
_NPAIR = 14.0


def _glcm_kernel(x_ref, o_ref):
    img = x_ref[0, 0]
    H, W = img.shape

    mn = jnp.min(img)
    mx = jnp.max(img)
    q = (_LEVELS - 1) * (img - mn) / (mx - mn) + 1.0
    r = jnp.round(q)
    mcf = (q == r).astype(jnp.float32)

    strip_sets = {
        (0, 5): ((slice(None), slice(W - 5, W)),),
        (-5, 5): ((slice(0, 5), slice(None)),
                  (slice(5, H), slice(W - 5, W))),
        (-5, 0): ((slice(0, 5), slice(None)),),
        (-5, -5): ((slice(0, 5), slice(None)),
                   (slice(5, H), slice(0, 5))),
    }

    def _strip_sum(p, strips):
        return sum(jnp.sum(p[rs, cs]) for rs, cs in strips)

    def _pad_sums(strips):
        s1 = s2 = 0.0
        for rs, cs in strips:
            rr = r[rs, cs]
            mm = mcf[rs, cs]
            s = 2.0 * rr - 20.0
            w2 = mm * (rr >= 4.0).astype(jnp.float32)
            s1 = s1 + jnp.sum(w2 * s * s)
            s2 = s2 + jnp.sum(w2)
        return s1, s2

    e = r * mcf
    e_up = jnp.roll(e, 5, axis=0)
    e_nbr = {
        (0, 5): jnp.roll(e, -5, axis=1),
        (-5, 5): jnp.roll(e_up, -5, axis=1),
        (-5, 0): e_up,
        (-5, -5): jnp.roll(e_up, 5, axis=1),
    }

    pad_cache = {s: _pad_sums((s,)) for strips in strip_sets.values()
                 for s in strips}

    feats = []
    for dy, dx in _OFFSETS:
        en = e_nbr[(dy, dx)]
        pair = jnp.where(en > 0.0, mcf, 0.0)
        d = r - en
        t = pair * d * d
        strips = strip_sets[(dy, dx)]
        s_num = jnp.sum(t) - _strip_sum(t, strips)
        s_cnt = jnp.sum(pair) - _strip_sum(pair, strips)
        spc1 = sum(pad_cache[s][0] for s in strips)
        spc2 = sum(pad_cache[s][1] for s in strips)
        pt = float(H * W - (H - abs(dy)) * (W - abs(dx)))
        num = 2.0 * s_num + _W1SUM * pt - 2.0 * spc1
        den = 2.0 * s_cnt + _NPAIR * pt - 2.0 * spc2
        feats.append(num / den)

    o_ref[:, :, :] = jnp.stack(feats).reshape(1, 1, len(_OFFSETS))


def kernel(x):
    B, C, H, W = x.shape
    out = pl.pallas_call(
        _glcm_kernel,
        grid=(B,),
        in_specs=[pl.BlockSpec((1, C, H, W), lambda i: (i, 0, 0, 0))],
        out_specs=pl.BlockSpec((1, 1, len(_OFFSETS)), lambda i: (i, 0, 0)),
        out_shape=jax.ShapeDtypeStruct((B, 1, len(_OFFSETS)), jnp.float32),
        compiler_params=pltpu.CompilerParams(
            dimension_semantics=("parallel",),
        ),
    )(x)
    return out.reshape(B, 1, 1, len(_OFFSETS))


# speedup vs baseline: 1.0384x; 1.0384x over previous
import jax
import jax.numpy as jnp
from jax.experimental import pallas as pl
from jax.experimental.pallas import tpu as pltpu

_LEVELS = 16
_OFFSETS = ((0, 5), (-5, 5), (-5, 0), (-5, -5))
_W1SUM = 728.0
_NPAIR = 14.0


def _glcm_kernel(x_ref, o_ref):
    img = x_ref[0, 0]
    H, W = img.shape

    mn = jnp.min(img)
    mx = jnp.max(img)
    q = (_LEVELS - 1) * (img - mn) / (mx - mn) + 1.0
    r = jnp.round(q)
    mcf = (q == r).astype(jnp.float32)

    strip_sets = {
        (0, 5): ((slice(None), slice(W - 5, W)),),
        (-5, 5): ((slice(0, 5), slice(None)),
                  (slice(5, H), slice(W - 5, W))),
        (-5, 0): ((slice(0, 5), slice(None)),),
        (-5, -5): ((slice(0, 5), slice(None)),
                   (slice(5, H), slice(0, 5))),
    }

    def _strip_sum(p, strips):
        return sum(jnp.sum(p[rs, cs]) for rs, cs in strips)

    def _pad_sums(strips):
        s1 = s2 = 0.0
        for rs, cs in strips:
            rr = r[rs, cs]
            mm = mcf[rs, cs]
            s = 2.0 * rr - 20.0
            w2 = mm * (rr >= 4.0).astype(jnp.float32)
            s1 = s1 + jnp.sum(w2 * s * s)
            s2 = s2 + jnp.sum(w2)
        return s1, s2

    e = r * mcf
    e_up = jnp.roll(e, 5, axis=0)
    e_nbr = {
        (0, 5): jnp.roll(e, -5, axis=1),
        (-5, 5): jnp.roll(e_up, -5, axis=1),
        (-5, 0): e_up,
        (-5, -5): jnp.roll(e_up, 5, axis=1),
    }

    feats = []
    for dy, dx in _OFFSETS:
        en = e_nbr[(dy, dx)]
        pair = jnp.where(en > 0.0, mcf, 0.0)
        d = r - en
        t = pair * d * d
        strips = strip_sets[(dy, dx)]
        s_num = jnp.sum(t) - _strip_sum(t, strips)
        s_cnt = jnp.sum(pair) - _strip_sum(pair, strips)
        spc1, spc2 = _pad_sums(strips)
        pt = float(H * W - (H - abs(dy)) * (W - abs(dx)))
        num = 2.0 * s_num + _W1SUM * pt - 2.0 * spc1
        den = 2.0 * s_cnt + _NPAIR * pt - 2.0 * spc2
        feats.append(num / den)

    o_ref[:, :, :] = jnp.stack(feats).reshape(1, 1, len(_OFFSETS))


def kernel(x):
    B, C, H, W = x.shape
    out = pl.pallas_call(
        _glcm_kernel,
        grid=(B,),
        in_specs=[pl.BlockSpec((1, C, H, W), lambda i: (i, 0, 0, 0))],
        out_specs=pl.BlockSpec((1, 1, len(_OFFSETS)), lambda i: (i, 0, 0)),
        out_shape=jax.ShapeDtypeStruct((B, 1, len(_OFFSETS)), jnp.float32),
        compiler_params=pltpu.CompilerParams(
            dimension_semantics=("parallel",),
        ),
    )(x)
    return out.reshape(B, 1, 1, len(_OFFSETS))
